# Initial kernel scaffold; baseline (speedup 1.0000x reference)
import jax
import jax.numpy as jnp
from jax.experimental import pallas as pl


def kernel(x, edge_index, edge_attr, u, batch, W_edge, b_edge, W_node, b_node, W_glob, b_glob, w_ih_e, w_hh_e, b_ih_e, b_hh_e, w_ih_n, w_hh_n, b_ih_n, b_hh_n, w_ih_g, w_hh_g, b_ih_g, b_hh_g):
    raise NotImplementedError("write your pallas kernel here")


# speedup vs baseline: 7.8152x; 7.8152x over previous
import functools

import jax
import jax.numpy as jnp
from jax import lax
from jax.experimental import pallas as pl
from jax.experimental.pallas import tpu as pltpu
from jax.experimental.pallas import tpu_sc as plsc

_N = 10000
_E = 320000
_XH = 128
_EH = 16
_UH = 128
_STEPS = 2

_NC = 2
_NS = 16
_NW = _NC * _NS

_T = 10240
_EP = _NW * _T
_RP = _EP // 128
_CHUNK = 2048
_KROW = _CHUNK // 128
_NCH = _T // _CHUNK
_TROW = _T // 128
_NP = 10240
_RPT = _NP // _NS

_BE = 512
_BN = 2000

def _sc_gather_body(srci, dsti, ps, pd, outs, outd, idx_a, idx_b, buf_s, buf_d,
                    sem):
    wid = lax.axis_index("s") * _NC + lax.axis_index("c")

    def chunk(k, carry):
        row0 = wid * _TROW + k * _KROW
        base = row0 * 128
        pltpu.sync_copy(srci.at[pl.ds(row0, _KROW)], idx_a)
        pltpu.sync_copy(dsti.at[pl.ds(row0, _KROW)], idx_b)
        cps = []
        for j in range(_KROW):
            cps.append(pltpu.async_copy(
                ps.at[idx_a.at[j]], buf_s.at[pl.ds(j * 128, 128)], sem))
            cps.append(pltpu.async_copy(
                pd.at[idx_b.at[j]], buf_d.at[pl.ds(j * 128, 128)], sem))
        for cp in cps:
            cp.wait()
        pltpu.sync_copy(buf_s, outs.at[pl.ds(base, _CHUNK)])
        pltpu.sync_copy(buf_d, outd.at[pl.ds(base, _CHUNK)])
        return carry

    lax.fori_loop(0, _NCH, chunk, 0)


@functools.cache
def _get_sc_gather():
    mesh = plsc.VectorSubcoreMesh(core_axis_name="c", subcore_axis_name="s",
                                  num_cores=_NC, num_subcores=_NS)
    return pl.kernel(
        _sc_gather_body,
        out_type=(jax.ShapeDtypeStruct((_EP, _EH), jnp.float32),
                  jax.ShapeDtypeStruct((_EP, _EH), jnp.float32)),
        mesh=mesh,
        scratch_types=[
            pltpu.VMEM((_KROW, 128), jnp.int32),
            pltpu.VMEM((_KROW, 128), jnp.int32),
            pltpu.VMEM((_CHUNK, _EH), jnp.float32),
            pltpu.VMEM((_CHUNK, _EH), jnp.float32),
            pltpu.SemaphoreType.DMA,
        ],
        compiler_params=pltpu.CompilerParams(use_tc_tiling_on_sc=False),
    )


def _sc_scatter_body(dsti, vals, out, idx_a, vbuf, zbuf, shared):
    c = lax.axis_index("c")
    s = lax.axis_index("s")
    wid = s * _NC + c

    def zrow(i, carry):
        zbuf[i] = jnp.zeros((_EH,), jnp.float32)
        return carry

    lax.fori_loop(0, _RPT, zrow, 0)
    pltpu.sync_copy(zbuf, shared.at[pl.ds(s * _RPT, _RPT)])
    plsc.subcore_barrier()

    def chunk(k, carry):
        row0 = wid * _TROW + k * _KROW
        base = row0 * 128
        pltpu.sync_copy(dsti.at[pl.ds(row0, _KROW)], idx_a)
        pltpu.sync_copy(vals.at[pl.ds(base, _CHUNK)], vbuf)
        for j in range(_KROW):
            pltpu.sync_copy(vbuf.at[pl.ds(j * 128, 128)],
                            shared.at[idx_a.at[j]], add=True)
        return carry

    lax.fori_loop(0, _NCH, chunk, 0)
    plsc.subcore_barrier()
    pltpu.sync_copy(shared.at[pl.ds(s * _RPT, _RPT)], zbuf)
    pltpu.sync_copy(zbuf, out.at[c, pl.ds(s * _RPT, _RPT)])


@functools.cache
def _get_sc_scatter():
    mesh = plsc.VectorSubcoreMesh(core_axis_name="c", subcore_axis_name="s",
                                  num_cores=_NC, num_subcores=_NS)
    return pl.kernel(
        _sc_scatter_body,
        out_type=jax.ShapeDtypeStruct((_NC, _NP, _EH), jnp.float32),
        mesh=mesh,
        scratch_types=[
            pltpu.VMEM((_KROW, 128), jnp.int32),
            pltpu.VMEM((_CHUNK, _EH), jnp.float32),
            pltpu.VMEM((_RPT, _EH), jnp.float32),
            pltpu.VMEM_SHARED((_NP, _EH), jnp.float32),
        ],
        compiler_params=pltpu.CompilerParams(use_tc_tiling_on_sc=False),
    )


def _edge_body(gs, gd, ea, k3, krih, kzih, knih, krhh, kzhh, knhh,
               ce, bre, bze, bine, bhne, out):
    a = ea[...]
    d = lambda m, w: jnp.dot(m, w[...], preferred_element_type=jnp.float32)
    eo = jnp.maximum(gs[...] + gd[...] + d(a, k3) + ce[...], 0.0)
    r = jax.nn.sigmoid(d(eo, krih) + d(a, krhh) + bre[...])
    z = jax.nn.sigmoid(d(eo, kzih) + d(a, kzhh) + bze[...])
    n = jnp.tanh(d(eo, knih) + bine[...] + r * (d(a, knhh) + bhne[...]))
    out[...] = (1.0 - z) * n + z * a


_edge = pl.pallas_call(
    _edge_body,
    grid=(_EP // 8 // _BE,),
    in_specs=[pl.BlockSpec((_BE, 128), lambda i: (i, 0))] * 3
    + [pl.BlockSpec((128, 128), lambda i: (0, 0))] * 7
    + [pl.BlockSpec((1, 128), lambda i: (0, 0))] * 5,
    out_specs=pl.BlockSpec((_BE, 128), lambda i: (i, 0)),
    out_shape=jax.ShapeDtypeStruct((_EP // 8, 128), jnp.float32),
)


def _node_body(x, aggp, cn, wn1, wn2, wihn, whhn, bihn, bhhn, xo_ref, sum_ref):
    xx = x[...]
    agg = aggp[0] + aggp[1]
    d = lambda m, w: jnp.dot(m, w[...], preferred_element_type=jnp.float32)
    xo = jnp.maximum(d(xx, wn1) + d(agg, wn2) + cn[...], 0.0)
    gi = d(xo, wihn) + bihn[...]
    gh = d(xx, whhn) + bhhn[...]
    r = jax.nn.sigmoid(gi[:, :_XH] + gh[:, :_XH])
    z = jax.nn.sigmoid(gi[:, _XH:2 * _XH] + gh[:, _XH:2 * _XH])
    n = jnp.tanh(gi[:, 2 * _XH:] + r * gh[:, 2 * _XH:])
    xn = (1.0 - z) * n + z * xx
    xo_ref[...] = xn

    @pl.when(pl.program_id(0) == 0)
    def _():
        sum_ref[...] = jnp.zeros_like(sum_ref)

    sum_ref[...] += jnp.sum(xn, axis=0, keepdims=True)


_node = pl.pallas_call(
    _node_body,
    grid=(_N // _BN,),
    in_specs=[
        pl.BlockSpec((_BN, 128), lambda i: (i, 0)),
        pl.BlockSpec((_NC, _BN, _EH), lambda i: (0, i, 0)),
        pl.BlockSpec((1, 128), lambda i: (0, 0)),
        pl.BlockSpec((128, 128), lambda i: (0, 0)),
        pl.BlockSpec((_EH, 128), lambda i: (0, 0)),
        pl.BlockSpec((128, 384), lambda i: (0, 0)),
        pl.BlockSpec((128, 384), lambda i: (0, 0)),
        pl.BlockSpec((1, 384), lambda i: (0, 0)),
        pl.BlockSpec((1, 384), lambda i: (0, 0)),
    ],
    out_specs=[
        pl.BlockSpec((_BN, 128), lambda i: (i, 0)),
        pl.BlockSpec((1, 128), lambda i: (0, 0)),
    ],
    out_shape=[
        jax.ShapeDtypeStruct((_N, 128), jnp.float32),
        jax.ShapeDtypeStruct((1, 128), jnp.float32),
    ],
)


def _prep_body(x, u, w12, w4, be, wn3, bn, ps_ref, pd_ref, ce_ref, cn_ref):
    d = lambda m, w: jnp.dot(m, w[...], preferred_element_type=jnp.float32)
    pp = d(x[...], w12)
    ps_ref[...] = pp[:, :_EH]
    pd_ref[...] = pp[:, _EH:]
    uu = u[...]
    ce_ref[...] = d(uu, w4) + be[...]
    cn_ref[...] = d(uu, wn3) + bn[...]


_prep = pl.pallas_call(
    _prep_body,
    out_shape=[
        jax.ShapeDtypeStruct((_N, _EH), jnp.float32),
        jax.ShapeDtypeStruct((_N, _EH), jnp.float32),
        jax.ShapeDtypeStruct((1, _EH), jnp.float32),
        jax.ShapeDtypeStruct((1, 128), jnp.float32),
    ],
)


def _glob_body(u, sumx, wg1, wg2, bg, wihg, whhg, bihg, bhhg,
               x, w12, w4, be, wn3, bn,
               un_ref, ps_ref, pd_ref, ce_ref, cn_ref):
    d = lambda m, w: jnp.dot(m, w[...], preferred_element_type=jnp.float32)
    uu = u[...]
    mean = sumx[...] * (1.0 / _N)
    uo = jnp.maximum(d(uu, wg1) + d(mean, wg2) + bg[...], 0.0)
    gi = d(uo, wihg) + bihg[...]
    gh = d(uu, whhg) + bhhg[...]
    r = jax.nn.sigmoid(gi[:, :_UH] + gh[:, :_UH])
    z = jax.nn.sigmoid(gi[:, _UH:2 * _UH] + gh[:, _UH:2 * _UH])
    n = jnp.tanh(gi[:, 2 * _UH:] + r * gh[:, 2 * _UH:])
    un = (1.0 - z) * n + z * uu
    un_ref[...] = un
    pp = d(x[...], w12)
    ps_ref[...] = pp[:, :_EH]
    pd_ref[...] = pp[:, _EH:]
    ce_ref[...] = d(un, w4) + be[...]
    cn_ref[...] = d(un, wn3) + bn[...]


_glob = pl.pallas_call(
    _glob_body,
    out_shape=[
        jax.ShapeDtypeStruct((1, 128), jnp.float32),
        jax.ShapeDtypeStruct((_N, _EH), jnp.float32),
        jax.ShapeDtypeStruct((_N, _EH), jnp.float32),
        jax.ShapeDtypeStruct((1, _EH), jnp.float32),
        jax.ShapeDtypeStruct((1, 128), jnp.float32),
    ],
)


def kernel(x, edge_index, edge_attr, u, batch, W_edge, b_edge, W_node, b_node,
           W_glob, b_glob, w_ih_e, w_hh_e, b_ih_e, b_hh_e, w_ih_n, w_hh_n,
           b_ih_n, b_hh_n, w_ih_g, w_hh_g, b_ih_g, b_hh_g):
    f32 = jnp.float32
    src = edge_index[0]
    dst = edge_index[1]
    pad = _EP - _E
    fill_g = jnp.arange(pad, dtype=jnp.int32) % _N
    src2 = jnp.concatenate([src, fill_g]).reshape(_RP, 128)
    dstg2 = jnp.concatenate([dst, fill_g]).reshape(_RP, 128)
    fill_s = _N + (jnp.arange(pad, dtype=jnp.int32) % (_NP - _N))
    dsts2 = jnp.concatenate([dst, fill_s]).reshape(_RP, 128)
    ea = jnp.concatenate([edge_attr, jnp.zeros((pad, _EH), f32)],
                         axis=0).reshape(_EP // 8, 128)

    W12T = jnp.concatenate([W_edge[:, :_XH].T, W_edge[:, _XH:2 * _XH].T],
                           axis=1)
    W3T = W_edge[:, 2 * _XH:2 * _XH + _EH].T
    W4T = W_edge[:, 2 * _XH + _EH:].T
    be = b_edge[None, :]
    Wn1T = W_node[:, :_XH].T
    Wn2T = W_node[:, _XH:_XH + _EH].T
    Wn3T = W_node[:, _XH + _EH:].T
    bn = b_node[None, :]
    Wg1T = W_glob[:, :_UH].T
    Wg2T = W_glob[:, _UH:].T
    bg = b_glob[None, :]

    eye8 = jnp.eye(8, dtype=f32)
    kr = lambda w: jnp.kron(eye8, w)
    K3 = kr(W3T)
    Krih = kr(w_ih_e[:_EH].T)
    Kzih = kr(w_ih_e[_EH:2 * _EH].T)
    Knih = kr(w_ih_e[2 * _EH:].T)
    Krhh = kr(w_hh_e[:_EH].T)
    Kzhh = kr(w_hh_e[_EH:2 * _EH].T)
    Knhh = kr(w_hh_e[2 * _EH:].T)
    bre = jnp.tile(b_ih_e[:_EH] + b_hh_e[:_EH], 8)[None]
    bze = jnp.tile(b_ih_e[_EH:2 * _EH] + b_hh_e[_EH:2 * _EH], 8)[None]
    bine = jnp.tile(b_ih_e[2 * _EH:], 8)[None]
    bhne = jnp.tile(b_hh_e[2 * _EH:], 8)[None]

    wihnT = w_ih_n.T
    whhnT = w_hh_n.T
    bihn = b_ih_n[None]
    bhhn = b_hh_n[None]
    wihgT = w_ih_g.T
    whhgT = w_hh_g.T
    bihg = b_ih_g[None]
    bhhg = b_hh_g[None]

    ps, pd, ce, cn = _prep(x, u, W12T, W4T, be, Wn3T, bn)
    xs = [x]
    us = [u]
    for _ in range(_STEPS):
        gs, gd = _get_sc_gather()(src2, dstg2, ps, pd)
        gs_p = gs.reshape(_EP // 8, 128)
        gd_p = gd.reshape(_EP // 8, 128)
        ce_t = jnp.tile(ce, (1, 8))
        ea = _edge(gs_p, gd_p, ea, K3, Krih, Kzih, Knih, Krhh, Kzhh, Knhh,
                   ce_t, bre, bze, bine, bhne)
        aggp = _get_sc_scatter()(dsts2, ea.reshape(_EP, _EH))
        x, sumx = _node(x, aggp, cn, Wn1T, Wn2T, wihnT, whhnT, bihn, bhhn)
        xs.append(x)
        u, ps, pd, ce, cn = _glob(u, sumx, Wg1T, Wg2T, bg, wihgT, whhgT,
                                  bihg, bhhg, x, W12T, W4T, be, Wn3T, bn)
        us.append(u)
    return jnp.concatenate(xs, axis=1), jnp.concatenate(us, axis=1)


# speedup vs baseline: 8.2776x; 1.0592x over previous
import functools

import jax
import jax.numpy as jnp
from jax import lax
from jax.experimental import pallas as pl
from jax.experimental.pallas import tpu as pltpu
from jax.experimental.pallas import tpu_sc as plsc

_N = 10000
_E = 320000
_XH = 128
_EH = 16
_UH = 128
_STEPS = 2

_NC = 2
_NS = 16
_NW = _NC * _NS

_T = 10240
_EP = _NW * _T
_RP = _EP // 128
_CHUNK = 2048
_KROW = _CHUNK // 128
_NCH = _T // _CHUNK
_TROW = _T // 128
_NP = 10240
_RPT = _NP // _NS

_BE = 512
_BN = 2000

def _sc_gather_body(srci, dsti, ps, pd, outs, outd, idx_a, idx_b, buf_s, buf_d,
                    tbuf, sh_ps, sh_pd, sem):
    c = lax.axis_index("c")
    s = lax.axis_index("s")
    wid = s * _NC + c

    rows = _N // _NS
    pltpu.sync_copy(ps.at[pl.ds(s * rows, rows)], tbuf)
    pltpu.sync_copy(tbuf, sh_ps.at[pl.ds(s * rows, rows)])
    pltpu.sync_copy(pd.at[pl.ds(s * rows, rows)], tbuf)
    pltpu.sync_copy(tbuf, sh_pd.at[pl.ds(s * rows, rows)])
    plsc.subcore_barrier()

    def chunk(k, carry):
        row0 = wid * _TROW + k * _KROW
        pltpu.sync_copy(srci.at[pl.ds(row0, _KROW)], idx_a)
        pltpu.sync_copy(dsti.at[pl.ds(row0, _KROW)], idx_b)
        cps = []
        for j in range(_KROW):
            cps.append(pltpu.async_copy(
                sh_ps.at[idx_a.at[j]], buf_s.at[pl.ds(j * 128, 128)], sem))
            cps.append(pltpu.async_copy(
                sh_pd.at[idx_b.at[j]], buf_d.at[pl.ds(j * 128, 128)], sem))
        for cp in cps:
            cp.wait()
        pltpu.sync_copy(buf_s, outs.at[pl.ds(row0 * 128, _CHUNK)])
        pltpu.sync_copy(buf_d, outd.at[pl.ds(row0 * 128, _CHUNK)])
        return carry

    lax.fori_loop(0, _NCH, chunk, 0)


@functools.cache
def _get_sc_gather():
    mesh = plsc.VectorSubcoreMesh(core_axis_name="c", subcore_axis_name="s",
                                  num_cores=_NC, num_subcores=_NS)
    return pl.kernel(
        _sc_gather_body,
        out_type=(jax.ShapeDtypeStruct((_EP, _EH), jnp.float32),
                  jax.ShapeDtypeStruct((_EP, _EH), jnp.float32)),
        mesh=mesh,
        scratch_types=[
            pltpu.VMEM((_KROW, 128), jnp.int32),
            pltpu.VMEM((_KROW, 128), jnp.int32),
            pltpu.VMEM((_CHUNK, _EH), jnp.float32),
            pltpu.VMEM((_CHUNK, _EH), jnp.float32),
            pltpu.VMEM((_N // _NS, _EH), jnp.float32),
            pltpu.VMEM_SHARED((_N, _EH), jnp.float32),
            pltpu.VMEM_SHARED((_N, _EH), jnp.float32),
            pltpu.SemaphoreType.DMA,
        ],
        compiler_params=pltpu.CompilerParams(use_tc_tiling_on_sc=False),
    )


def _sc_scatter_body(dsti, vals, out, idx_a, vbuf, zbuf, shared, sem):
    c = lax.axis_index("c")
    s = lax.axis_index("s")
    wid = s * _NC + c

    def zrow(i, carry):
        zbuf[i] = jnp.zeros((_EH,), jnp.float32)
        return carry

    lax.fori_loop(0, _RPT, zrow, 0)
    pltpu.sync_copy(zbuf, shared.at[pl.ds(s * _RPT, _RPT)])
    plsc.subcore_barrier()

    def chunk(k, carry):
        row0 = wid * _TROW + k * _KROW
        pltpu.sync_copy(dsti.at[pl.ds(row0, _KROW)], idx_a)
        pltpu.sync_copy(vals.at[pl.ds(row0 * 128, _CHUNK)], vbuf)
        cps = []
        for j in range(_KROW):
            cps.append(pltpu.async_copy(vbuf.at[pl.ds(j * 128, 128)],
                                        shared.at[idx_a.at[j]], sem,
                                        add=True))
        for cp in cps:
            cp.wait()
        return carry

    lax.fori_loop(0, _NCH, chunk, 0)
    plsc.subcore_barrier()
    pltpu.sync_copy(shared.at[pl.ds(s * _RPT, _RPT)], zbuf)
    pltpu.sync_copy(zbuf, out.at[c, pl.ds(s * _RPT, _RPT)])


@functools.cache
def _get_sc_scatter():
    mesh = plsc.VectorSubcoreMesh(core_axis_name="c", subcore_axis_name="s",
                                  num_cores=_NC, num_subcores=_NS)
    return pl.kernel(
        _sc_scatter_body,
        out_type=jax.ShapeDtypeStruct((_NC, _NP, _EH), jnp.float32),
        mesh=mesh,
        scratch_types=[
            pltpu.VMEM((_KROW, 128), jnp.int32),
            pltpu.VMEM((_CHUNK, _EH), jnp.float32),
            pltpu.VMEM((_RPT, _EH), jnp.float32),
            pltpu.VMEM_SHARED((_NP, _EH), jnp.float32),
            pltpu.SemaphoreType.DMA,
        ],
        compiler_params=pltpu.CompilerParams(use_tc_tiling_on_sc=False),
    )


def _edge_body(gs, gd, ea, k3, krih, kzih, knih, krhh, kzhh, knhh,
               ce, bre, bze, bine, bhne, out):
    a = ea[...]
    d = lambda m, w: jnp.dot(m, w[...], preferred_element_type=jnp.float32)
    eo = jnp.maximum(gs[...] + gd[...] + d(a, k3) + ce[...], 0.0)
    r = jax.nn.sigmoid(d(eo, krih) + d(a, krhh) + bre[...])
    z = jax.nn.sigmoid(d(eo, kzih) + d(a, kzhh) + bze[...])
    n = jnp.tanh(d(eo, knih) + bine[...] + r * (d(a, knhh) + bhne[...]))
    out[...] = (1.0 - z) * n + z * a


_edge = pl.pallas_call(
    _edge_body,
    grid=(_EP // 8 // _BE,),
    in_specs=[pl.BlockSpec((_BE, 128), lambda i: (i, 0))] * 3
    + [pl.BlockSpec((128, 128), lambda i: (0, 0))] * 7
    + [pl.BlockSpec((1, 128), lambda i: (0, 0))] * 5,
    out_specs=pl.BlockSpec((_BE, 128), lambda i: (i, 0)),
    out_shape=jax.ShapeDtypeStruct((_EP // 8, 128), jnp.float32),
)


def _node_body(x, aggp, cn, wn1, wn2, wihn, whhn, bihn, bhhn, xo_ref, sum_ref):
    xx = x[...]
    agg = aggp[0] + aggp[1]
    d = lambda m, w: jnp.dot(m, w[...], preferred_element_type=jnp.float32)
    xo = jnp.maximum(d(xx, wn1) + d(agg, wn2) + cn[...], 0.0)
    gi = d(xo, wihn) + bihn[...]
    gh = d(xx, whhn) + bhhn[...]
    r = jax.nn.sigmoid(gi[:, :_XH] + gh[:, :_XH])
    z = jax.nn.sigmoid(gi[:, _XH:2 * _XH] + gh[:, _XH:2 * _XH])
    n = jnp.tanh(gi[:, 2 * _XH:] + r * gh[:, 2 * _XH:])
    xn = (1.0 - z) * n + z * xx
    xo_ref[...] = xn

    @pl.when(pl.program_id(0) == 0)
    def _():
        sum_ref[...] = jnp.zeros_like(sum_ref)

    sum_ref[...] += jnp.sum(xn, axis=0, keepdims=True)


_node = pl.pallas_call(
    _node_body,
    grid=(_N // _BN,),
    in_specs=[
        pl.BlockSpec((_BN, 128), lambda i: (i, 0)),
        pl.BlockSpec((_NC, _BN, _EH), lambda i: (0, i, 0)),
        pl.BlockSpec((1, 128), lambda i: (0, 0)),
        pl.BlockSpec((128, 128), lambda i: (0, 0)),
        pl.BlockSpec((_EH, 128), lambda i: (0, 0)),
        pl.BlockSpec((128, 384), lambda i: (0, 0)),
        pl.BlockSpec((128, 384), lambda i: (0, 0)),
        pl.BlockSpec((1, 384), lambda i: (0, 0)),
        pl.BlockSpec((1, 384), lambda i: (0, 0)),
    ],
    out_specs=[
        pl.BlockSpec((_BN, 128), lambda i: (i, 0)),
        pl.BlockSpec((1, 128), lambda i: (0, 0)),
    ],
    out_shape=[
        jax.ShapeDtypeStruct((_N, 128), jnp.float32),
        jax.ShapeDtypeStruct((1, 128), jnp.float32),
    ],
)


def _prep_body(x, u, w12, w4, be, wn3, bn, ps_ref, pd_ref, ce_ref, cn_ref):
    d = lambda m, w: jnp.dot(m, w[...], preferred_element_type=jnp.float32)
    pp = d(x[...], w12)
    ps_ref[...] = pp[:, :_EH]
    pd_ref[...] = pp[:, _EH:]
    uu = u[...]
    ce_ref[...] = d(uu, w4) + be[...]
    cn_ref[...] = d(uu, wn3) + bn[...]


_prep = pl.pallas_call(
    _prep_body,
    out_shape=[
        jax.ShapeDtypeStruct((_N, _EH), jnp.float32),
        jax.ShapeDtypeStruct((_N, _EH), jnp.float32),
        jax.ShapeDtypeStruct((1, _EH), jnp.float32),
        jax.ShapeDtypeStruct((1, 128), jnp.float32),
    ],
)


def _glob_body(u, sumx, wg1, wg2, bg, wihg, whhg, bihg, bhhg,
               x, w12, w4, be, wn3, bn,
               un_ref, ps_ref, pd_ref, ce_ref, cn_ref):
    d = lambda m, w: jnp.dot(m, w[...], preferred_element_type=jnp.float32)
    uu = u[...]
    mean = sumx[...] * (1.0 / _N)
    uo = jnp.maximum(d(uu, wg1) + d(mean, wg2) + bg[...], 0.0)
    gi = d(uo, wihg) + bihg[...]
    gh = d(uu, whhg) + bhhg[...]
    r = jax.nn.sigmoid(gi[:, :_UH] + gh[:, :_UH])
    z = jax.nn.sigmoid(gi[:, _UH:2 * _UH] + gh[:, _UH:2 * _UH])
    n = jnp.tanh(gi[:, 2 * _UH:] + r * gh[:, 2 * _UH:])
    un = (1.0 - z) * n + z * uu
    un_ref[...] = un
    pp = d(x[...], w12)
    ps_ref[...] = pp[:, :_EH]
    pd_ref[...] = pp[:, _EH:]
    ce_ref[...] = d(un, w4) + be[...]
    cn_ref[...] = d(un, wn3) + bn[...]


_glob = pl.pallas_call(
    _glob_body,
    out_shape=[
        jax.ShapeDtypeStruct((1, 128), jnp.float32),
        jax.ShapeDtypeStruct((_N, _EH), jnp.float32),
        jax.ShapeDtypeStruct((_N, _EH), jnp.float32),
        jax.ShapeDtypeStruct((1, _EH), jnp.float32),
        jax.ShapeDtypeStruct((1, 128), jnp.float32),
    ],
)


def kernel(x, edge_index, edge_attr, u, batch, W_edge, b_edge, W_node, b_node,
           W_glob, b_glob, w_ih_e, w_hh_e, b_ih_e, b_hh_e, w_ih_n, w_hh_n,
           b_ih_n, b_hh_n, w_ih_g, w_hh_g, b_ih_g, b_hh_g):
    f32 = jnp.float32
    src = edge_index[0]
    dst = edge_index[1]
    pad = _EP - _E
    fill_g = jnp.arange(pad, dtype=jnp.int32) % _N
    src2 = jnp.concatenate([src, fill_g]).reshape(_RP, 128)
    dstg2 = jnp.concatenate([dst, fill_g]).reshape(_RP, 128)
    fill_s = _N + (jnp.arange(pad, dtype=jnp.int32) % (_NP - _N))
    dsts2 = jnp.concatenate([dst, fill_s]).reshape(_RP, 128)
    ea = jnp.concatenate([edge_attr, jnp.zeros((pad, _EH), f32)],
                         axis=0).reshape(_EP // 8, 128)

    W12T = jnp.concatenate([W_edge[:, :_XH].T, W_edge[:, _XH:2 * _XH].T],
                           axis=1)
    W3T = W_edge[:, 2 * _XH:2 * _XH + _EH].T
    W4T = W_edge[:, 2 * _XH + _EH:].T
    be = b_edge[None, :]
    Wn1T = W_node[:, :_XH].T
    Wn2T = W_node[:, _XH:_XH + _EH].T
    Wn3T = W_node[:, _XH + _EH:].T
    bn = b_node[None, :]
    Wg1T = W_glob[:, :_UH].T
    Wg2T = W_glob[:, _UH:].T
    bg = b_glob[None, :]

    eye8 = jnp.eye(8, dtype=f32)
    kr = lambda w: jnp.kron(eye8, w)
    K3 = kr(W3T)
    Krih = kr(w_ih_e[:_EH].T)
    Kzih = kr(w_ih_e[_EH:2 * _EH].T)
    Knih = kr(w_ih_e[2 * _EH:].T)
    Krhh = kr(w_hh_e[:_EH].T)
    Kzhh = kr(w_hh_e[_EH:2 * _EH].T)
    Knhh = kr(w_hh_e[2 * _EH:].T)
    bre = jnp.tile(b_ih_e[:_EH] + b_hh_e[:_EH], 8)[None]
    bze = jnp.tile(b_ih_e[_EH:2 * _EH] + b_hh_e[_EH:2 * _EH], 8)[None]
    bine = jnp.tile(b_ih_e[2 * _EH:], 8)[None]
    bhne = jnp.tile(b_hh_e[2 * _EH:], 8)[None]

    wihnT = w_ih_n.T
    whhnT = w_hh_n.T
    bihn = b_ih_n[None]
    bhhn = b_hh_n[None]
    wihgT = w_ih_g.T
    whhgT = w_hh_g.T
    bihg = b_ih_g[None]
    bhhg = b_hh_g[None]

    ps, pd, ce, cn = _prep(x, u, W12T, W4T, be, Wn3T, bn)
    xs = [x]
    us = [u]
    for _ in range(_STEPS):
        gs, gd = _get_sc_gather()(src2, dstg2, ps, pd)
        gs_p = gs.reshape(_EP // 8, 128)
        gd_p = gd.reshape(_EP // 8, 128)
        ce_t = jnp.tile(ce, (1, 8))
        ea = _edge(gs_p, gd_p, ea, K3, Krih, Kzih, Knih, Krhh, Kzhh, Knhh,
                   ce_t, bre, bze, bine, bhne)
        aggp = _get_sc_scatter()(dsts2, ea.reshape(_EP, _EH))
        x, sumx = _node(x, aggp, cn, Wn1T, Wn2T, wihnT, whhnT, bihn, bhhn)
        xs.append(x)
        u, ps, pd, ce, cn = _glob(u, sumx, Wg1T, Wg2T, bg, wihgT, whhgT,
                                  bihg, bhhg, x, W12T, W4T, be, Wn3T, bn)
        us.append(u)
    return jnp.concatenate(xs, axis=1), jnp.concatenate(us, axis=1)


# speedup vs baseline: 9.7612x; 1.1792x over previous
import functools

import jax
import jax.numpy as jnp
from jax import lax
from jax.experimental import pallas as pl
from jax.experimental.pallas import tpu as pltpu
from jax.experimental.pallas import tpu_sc as plsc

_N = 10000
_E = 320000
_XH = 128
_EH = 16
_UH = 128
_STEPS = 2

_NC = 2
_NS = 16
_NW = _NC * _NS

_T = 10240
_EP = _NW * _T
_RP = _EP // 128
_CHUNK = 2048
_KROW = _CHUNK // 128
_NCH = _T // _CHUNK
_TROW = _T // 128
_NP = 10240
_RPT = _NP // _NS

_BE = 512
_BN = 2000

def _sc_gather_body(srci, dsti, ps, pd, outs, outd, idx_a, idx_b, buf_s, buf_d,
                    tbuf, sh_ps, sh_pd, sem):
    c = lax.axis_index("c")
    s = lax.axis_index("s")
    wid = s * _NC + c

    rows = _N // _NS
    pltpu.sync_copy(ps.at[pl.ds(s * rows, rows)], tbuf)
    pltpu.sync_copy(tbuf, sh_ps.at[pl.ds(s * rows, rows)])
    pltpu.sync_copy(pd.at[pl.ds(s * rows, rows)], tbuf)
    pltpu.sync_copy(tbuf, sh_pd.at[pl.ds(s * rows, rows)])
    plsc.subcore_barrier()

    def chunk(k, carry):
        row0 = wid * _TROW + k * _KROW
        pltpu.sync_copy(srci.at[pl.ds(row0, _KROW)], idx_a)
        pltpu.sync_copy(dsti.at[pl.ds(row0, _KROW)], idx_b)
        cps = []
        for j in range(_KROW):
            cps.append(pltpu.async_copy(
                sh_ps.at[idx_a.at[j]], buf_s.at[pl.ds(j * 128, 128)], sem))
            cps.append(pltpu.async_copy(
                sh_pd.at[idx_b.at[j]], buf_d.at[pl.ds(j * 128, 128)], sem))
        for cp in cps:
            cp.wait()
        pltpu.sync_copy(buf_s, outs.at[pl.ds(row0 * 128, _CHUNK)])
        pltpu.sync_copy(buf_d, outd.at[pl.ds(row0 * 128, _CHUNK)])
        return carry

    lax.fori_loop(0, _NCH, chunk, 0)


@functools.cache
def _get_sc_gather():
    mesh = plsc.VectorSubcoreMesh(core_axis_name="c", subcore_axis_name="s",
                                  num_cores=_NC, num_subcores=_NS)
    return pl.kernel(
        _sc_gather_body,
        out_type=(jax.ShapeDtypeStruct((_EP, _EH), jnp.float32),
                  jax.ShapeDtypeStruct((_EP, _EH), jnp.float32)),
        mesh=mesh,
        scratch_types=[
            pltpu.VMEM((_KROW, 128), jnp.int32),
            pltpu.VMEM((_KROW, 128), jnp.int32),
            pltpu.VMEM((_CHUNK, _EH), jnp.float32),
            pltpu.VMEM((_CHUNK, _EH), jnp.float32),
            pltpu.VMEM((_N // _NS, _EH), jnp.float32),
            pltpu.VMEM_SHARED((_N, _EH), jnp.float32),
            pltpu.VMEM_SHARED((_N, _EH), jnp.float32),
            pltpu.SemaphoreType.DMA,
        ],
        compiler_params=pltpu.CompilerParams(use_tc_tiling_on_sc=False),
    )


def _sc_scatter_body(dsti, vals, out, idx_a, vbuf, zbuf, shared, sem):
    c = lax.axis_index("c")
    s = lax.axis_index("s")
    wid = s * _NC + c

    def zrow(i, carry):
        zbuf[i] = jnp.zeros((_EH,), jnp.float32)
        return carry

    lax.fori_loop(0, _RPT, zrow, 0)
    pltpu.sync_copy(zbuf, shared.at[pl.ds(s * _RPT, _RPT)])
    plsc.subcore_barrier()

    def chunk(k, carry):
        row0 = wid * _TROW + k * _KROW
        pltpu.sync_copy(dsti.at[pl.ds(row0, _KROW)], idx_a)
        pltpu.sync_copy(vals.at[pl.ds(row0 * 128, _CHUNK)], vbuf)
        cps = []
        for j in range(_KROW):
            cps.append(pltpu.async_copy(vbuf.at[pl.ds(j * 128, 128)],
                                        shared.at[idx_a.at[j]], sem,
                                        add=True))
        for cp in cps:
            cp.wait()
        return carry

    lax.fori_loop(0, _NCH, chunk, 0)
    plsc.subcore_barrier()
    pltpu.sync_copy(shared.at[pl.ds(s * _RPT, _RPT)], zbuf)
    pltpu.sync_copy(zbuf, out.at[c, pl.ds(s * _RPT, _RPT)])


@functools.cache
def _get_sc_scatter():
    mesh = plsc.VectorSubcoreMesh(core_axis_name="c", subcore_axis_name="s",
                                  num_cores=_NC, num_subcores=_NS)
    return pl.kernel(
        _sc_scatter_body,
        out_type=jax.ShapeDtypeStruct((_NC, _NP, _EH), jnp.float32),
        mesh=mesh,
        scratch_types=[
            pltpu.VMEM((_KROW, 128), jnp.int32),
            pltpu.VMEM((_CHUNK, _EH), jnp.float32),
            pltpu.VMEM((_RPT, _EH), jnp.float32),
            pltpu.VMEM_SHARED((_NP, _EH), jnp.float32),
            pltpu.SemaphoreType.DMA,
        ],
        compiler_params=pltpu.CompilerParams(use_tc_tiling_on_sc=False),
    )


def _edge_body(gs, gd, ea, k3, kih, khh, ce, bre, bze, bine, bhne, out):
    a = ea[...]
    d = lambda m, w: jnp.dot(m, w[...], preferred_element_type=jnp.float32)
    eo = jnp.maximum(gs[...] + gd[...] + d(a, k3) + ce[...], 0.0)
    gi = d(eo, kih)
    gh = d(a, khh)
    r = jax.nn.sigmoid(gi[:, :128] + gh[:, :128] + bre[...])
    z = jax.nn.sigmoid(gi[:, 128:256] + gh[:, 128:256] + bze[...])
    n = jnp.tanh(gi[:, 256:] + bine[...] + r * (gh[:, 256:] + bhne[...]))
    out[...] = (1.0 - z) * n + z * a


_edge = pl.pallas_call(
    _edge_body,
    grid=(_EP // 8 // _BE,),
    in_specs=[pl.BlockSpec((_BE, 128), lambda i: (i, 0))] * 3
    + [pl.BlockSpec((128, 128), lambda i: (0, 0))]
    + [pl.BlockSpec((128, 384), lambda i: (0, 0))] * 2
    + [pl.BlockSpec((1, 128), lambda i: (0, 0))] * 5,
    out_specs=pl.BlockSpec((_BE, 128), lambda i: (i, 0)),
    out_shape=jax.ShapeDtypeStruct((_EP // 8, 128), jnp.float32),
)


def _node_body(x, aggp, cn, wn1, wn2, wihn, whhn, bihn, bhhn, xo_ref, sum_ref):
    xx = x[...]
    agg = aggp[0] + aggp[1]
    d = lambda m, w: jnp.dot(m, w[...], preferred_element_type=jnp.float32)
    xo = jnp.maximum(d(xx, wn1) + d(agg, wn2) + cn[...], 0.0)
    gi = d(xo, wihn) + bihn[...]
    gh = d(xx, whhn) + bhhn[...]
    r = jax.nn.sigmoid(gi[:, :_XH] + gh[:, :_XH])
    z = jax.nn.sigmoid(gi[:, _XH:2 * _XH] + gh[:, _XH:2 * _XH])
    n = jnp.tanh(gi[:, 2 * _XH:] + r * gh[:, 2 * _XH:])
    xn = (1.0 - z) * n + z * xx
    xo_ref[...] = xn

    @pl.when(pl.program_id(0) == 0)
    def _():
        sum_ref[...] = jnp.zeros_like(sum_ref)

    sum_ref[...] += jnp.sum(xn, axis=0, keepdims=True)


_node = pl.pallas_call(
    _node_body,
    grid=(_N // _BN,),
    in_specs=[
        pl.BlockSpec((_BN, 128), lambda i: (i, 0)),
        pl.BlockSpec((_NC, _BN, _EH), lambda i: (0, i, 0)),
        pl.BlockSpec((1, 128), lambda i: (0, 0)),
        pl.BlockSpec((128, 128), lambda i: (0, 0)),
        pl.BlockSpec((_EH, 128), lambda i: (0, 0)),
        pl.BlockSpec((128, 384), lambda i: (0, 0)),
        pl.BlockSpec((128, 384), lambda i: (0, 0)),
        pl.BlockSpec((1, 384), lambda i: (0, 0)),
        pl.BlockSpec((1, 384), lambda i: (0, 0)),
    ],
    out_specs=[
        pl.BlockSpec((_BN, 128), lambda i: (i, 0)),
        pl.BlockSpec((1, 128), lambda i: (0, 0)),
    ],
    out_shape=[
        jax.ShapeDtypeStruct((_N, 128), jnp.float32),
        jax.ShapeDtypeStruct((1, 128), jnp.float32),
    ],
)


def _prep_body(x, u, w12, w4, be, wn3, bn, ps_ref, pd_ref, ce_ref, cn_ref):
    d = lambda m, w: jnp.dot(m, w[...], preferred_element_type=jnp.float32)
    pp = d(x[...], w12)
    ps_ref[...] = pp[:, :_EH]
    pd_ref[...] = pp[:, _EH:]
    uu = u[...]
    ce_ref[...] = d(uu, w4) + be[...]
    cn_ref[...] = d(uu, wn3) + bn[...]


_prep = pl.pallas_call(
    _prep_body,
    out_shape=[
        jax.ShapeDtypeStruct((_N, _EH), jnp.float32),
        jax.ShapeDtypeStruct((_N, _EH), jnp.float32),
        jax.ShapeDtypeStruct((1, _EH), jnp.float32),
        jax.ShapeDtypeStruct((1, 128), jnp.float32),
    ],
)


def _glob_body(u, sumx, wg1, wg2, bg, wihg, whhg, bihg, bhhg,
               x, w12, w4, be, wn3, bn,
               un_ref, ps_ref, pd_ref, ce_ref, cn_ref):
    d = lambda m, w: jnp.dot(m, w[...], preferred_element_type=jnp.float32)
    uu = u[...]
    mean = sumx[...] * (1.0 / _N)
    uo = jnp.maximum(d(uu, wg1) + d(mean, wg2) + bg[...], 0.0)
    gi = d(uo, wihg) + bihg[...]
    gh = d(uu, whhg) + bhhg[...]
    r = jax.nn.sigmoid(gi[:, :_UH] + gh[:, :_UH])
    z = jax.nn.sigmoid(gi[:, _UH:2 * _UH] + gh[:, _UH:2 * _UH])
    n = jnp.tanh(gi[:, 2 * _UH:] + r * gh[:, 2 * _UH:])
    un = (1.0 - z) * n + z * uu
    un_ref[...] = un
    pp = d(x[...], w12)
    ps_ref[...] = pp[:, :_EH]
    pd_ref[...] = pp[:, _EH:]
    ce_ref[...] = d(un, w4) + be[...]
    cn_ref[...] = d(un, wn3) + bn[...]


_glob = pl.pallas_call(
    _glob_body,
    out_shape=[
        jax.ShapeDtypeStruct((1, 128), jnp.float32),
        jax.ShapeDtypeStruct((_N, _EH), jnp.float32),
        jax.ShapeDtypeStruct((_N, _EH), jnp.float32),
        jax.ShapeDtypeStruct((1, _EH), jnp.float32),
        jax.ShapeDtypeStruct((1, 128), jnp.float32),
    ],
)


def kernel(x, edge_index, edge_attr, u, batch, W_edge, b_edge, W_node, b_node,
           W_glob, b_glob, w_ih_e, w_hh_e, b_ih_e, b_hh_e, w_ih_n, w_hh_n,
           b_ih_n, b_hh_n, w_ih_g, w_hh_g, b_ih_g, b_hh_g):
    f32 = jnp.float32
    src = edge_index[0]
    dst = edge_index[1]
    pad = _EP - _E
    fill_g = jnp.arange(pad, dtype=jnp.int32) % _N
    src2 = jnp.concatenate([src, fill_g]).reshape(_RP, 128)
    dstg2 = jnp.concatenate([dst, fill_g]).reshape(_RP, 128)
    fill_s = _N + (jnp.arange(pad, dtype=jnp.int32) % (_NP - _N))
    dsts2 = jnp.concatenate([dst, fill_s]).reshape(_RP, 128)
    ea = jnp.concatenate([edge_attr.reshape(_E // 8, 128),
                          jnp.zeros((pad // 8, 128), f32)], axis=0)

    W12T = jnp.concatenate([W_edge[:, :_XH].T, W_edge[:, _XH:2 * _XH].T],
                           axis=1)
    W3T = W_edge[:, 2 * _XH:2 * _XH + _EH].T
    W4T = W_edge[:, 2 * _XH + _EH:].T
    be = b_edge[None, :]
    Wn1T = W_node[:, :_XH].T
    Wn2T = W_node[:, _XH:_XH + _EH].T
    Wn3T = W_node[:, _XH + _EH:].T
    bn = b_node[None, :]
    Wg1T = W_glob[:, :_UH].T
    Wg2T = W_glob[:, _UH:].T
    bg = b_glob[None, :]

    eye8 = jnp.eye(8, dtype=f32)
    kr = lambda w: jnp.kron(eye8, w)
    K3 = kr(W3T)
    KIH = jnp.concatenate([kr(w_ih_e[:_EH].T), kr(w_ih_e[_EH:2 * _EH].T),
                           kr(w_ih_e[2 * _EH:].T)], axis=1)
    KHH = jnp.concatenate([kr(w_hh_e[:_EH].T), kr(w_hh_e[_EH:2 * _EH].T),
                           kr(w_hh_e[2 * _EH:].T)], axis=1)
    bre = jnp.tile(b_ih_e[:_EH] + b_hh_e[:_EH], 8)[None]
    bze = jnp.tile(b_ih_e[_EH:2 * _EH] + b_hh_e[_EH:2 * _EH], 8)[None]
    bine = jnp.tile(b_ih_e[2 * _EH:], 8)[None]
    bhne = jnp.tile(b_hh_e[2 * _EH:], 8)[None]

    wihnT = w_ih_n.T
    whhnT = w_hh_n.T
    bihn = b_ih_n[None]
    bhhn = b_hh_n[None]
    wihgT = w_ih_g.T
    whhgT = w_hh_g.T
    bihg = b_ih_g[None]
    bhhg = b_hh_g[None]

    ps, pd, ce, cn = _prep(x, u, W12T, W4T, be, Wn3T, bn)
    xs = [x]
    us = [u]
    for _ in range(_STEPS):
        gs, gd = _get_sc_gather()(src2, dstg2, ps, pd)
        gs_p = gs.reshape(_EP // 8, 128)
        gd_p = gd.reshape(_EP // 8, 128)
        ce_t = jnp.tile(ce, (1, 8))
        ea = _edge(gs_p, gd_p, ea, K3, KIH, KHH, ce_t, bre, bze, bine, bhne)
        aggp = _get_sc_scatter()(dsts2, ea.reshape(_EP, _EH))
        x, sumx = _node(x, aggp, cn, Wn1T, Wn2T, wihnT, whhnT, bihn, bhhn)
        xs.append(x)
        u, ps, pd, ce, cn = _glob(u, sumx, Wg1T, Wg2T, bg, wihgT, whhgT,
                                  bihg, bhhg, x, W12T, W4T, be, Wn3T, bn)
        us.append(u)
    return jnp.concatenate(xs, axis=1), jnp.concatenate(us, axis=1)


# speedup vs baseline: 9.8640x; 1.0105x over previous
import functools

import jax
import jax.numpy as jnp
from jax import lax
from jax.experimental import pallas as pl
from jax.experimental.pallas import tpu as pltpu
from jax.experimental.pallas import tpu_sc as plsc

_N = 10000
_E = 320000
_XH = 128
_EH = 16
_UH = 128
_STEPS = 2

_NC = 2
_NS = 16
_NW = _NC * _NS

_T = 10240
_EP = _NW * _T
_RP = _EP // 128
_CHUNK = 1024
_KROW = _CHUNK // 128
_NCH = _T // _CHUNK
_TROW = _T // 128
_NP = 10240
_RPT = _NP // _NS

_BE = 512
_BN = 2000

def _sc_gather_body(srci, dsti, ps, pd, outs, outd, idx_a, idx_b, buf_s, buf_d,
                    tbuf, sh_ps, sh_pd, gsem, wsem):
    c = lax.axis_index("c")
    s = lax.axis_index("s")
    wid = s * _NC + c

    rows = _N // _NS
    pltpu.sync_copy(ps.at[pl.ds(s * rows, rows)], tbuf)
    pltpu.sync_copy(tbuf, sh_ps.at[pl.ds(s * rows, rows)])
    pltpu.sync_copy(pd.at[pl.ds(s * rows, rows)], tbuf)
    pltpu.sync_copy(tbuf, sh_pd.at[pl.ds(s * rows, rows)])
    plsc.subcore_barrier()

    def fire(k):
        p = k % 2
        row0 = wid * _TROW + k * _KROW
        pltpu.sync_copy(srci.at[pl.ds(row0, _KROW)], idx_a.at[p])
        pltpu.sync_copy(dsti.at[pl.ds(row0, _KROW)], idx_b.at[p])
        cps = []
        for j in range(_KROW):
            cps.append(pltpu.async_copy(
                sh_ps.at[idx_a.at[p, j]],
                buf_s.at[p, pl.ds(j * 128, 128)], gsem[p]))
            cps.append(pltpu.async_copy(
                sh_pd.at[idx_b.at[p, j]],
                buf_d.at[p, pl.ds(j * 128, 128)], gsem[p]))
        return cps

    pend = {0: fire(0)}
    wr = {}
    for k in range(_NCH):
        p = k % 2
        q = (k + 1) % 2
        for cp in pend.pop(k):
            cp.wait()
        row0 = wid * _TROW + k * _KROW
        wr[k] = [
            pltpu.async_copy(buf_s.at[p], outs.at[pl.ds(row0 * 128, _CHUNK)],
                             wsem[p]),
            pltpu.async_copy(buf_d.at[p], outd.at[pl.ds(row0 * 128, _CHUNK)],
                             wsem[p]),
        ]
        if k + 1 < _NCH:
            if k - 1 >= 0:
                for cp in wr.pop(k - 1):
                    cp.wait()
            pend[k + 1] = fire(k + 1)
    for k in list(wr):
        for cp in wr.pop(k):
            cp.wait()


@functools.cache
def _get_sc_gather():
    mesh = plsc.VectorSubcoreMesh(core_axis_name="c", subcore_axis_name="s",
                                  num_cores=_NC, num_subcores=_NS)
    return pl.kernel(
        _sc_gather_body,
        out_type=(jax.ShapeDtypeStruct((_EP, _EH), jnp.float32),
                  jax.ShapeDtypeStruct((_EP, _EH), jnp.float32)),
        mesh=mesh,
        scratch_types=[
            pltpu.VMEM((2, _KROW, 128), jnp.int32),
            pltpu.VMEM((2, _KROW, 128), jnp.int32),
            pltpu.VMEM((2, _CHUNK, _EH), jnp.float32),
            pltpu.VMEM((2, _CHUNK, _EH), jnp.float32),
            pltpu.VMEM((_N // _NS, _EH), jnp.float32),
            pltpu.VMEM_SHARED((_N, _EH), jnp.float32),
            pltpu.VMEM_SHARED((_N, _EH), jnp.float32),
            (pltpu.SemaphoreType.DMA, pltpu.SemaphoreType.DMA),
            (pltpu.SemaphoreType.DMA, pltpu.SemaphoreType.DMA),
        ],
        compiler_params=pltpu.CompilerParams(use_tc_tiling_on_sc=False),
    )


def _sc_scatter_body(dsti, vals, out, idx_a, vbuf, zbuf, shared, lsem, ssem):
    c = lax.axis_index("c")
    s = lax.axis_index("s")
    wid = s * _NC + c

    def zrow(i, carry):
        zbuf[i] = jnp.zeros((_EH,), jnp.float32)
        return carry

    lax.fori_loop(0, _RPT, zrow, 0)
    pltpu.sync_copy(zbuf, shared.at[pl.ds(s * _RPT, _RPT)])
    plsc.subcore_barrier()

    def fire_load(k):
        p = k % 2
        row0 = wid * _TROW + k * _KROW
        return [
            pltpu.async_copy(dsti.at[pl.ds(row0, _KROW)], idx_a.at[p],
                             lsem[p]),
            pltpu.async_copy(vals.at[pl.ds(row0 * 128, _CHUNK)], vbuf.at[p],
                             lsem[p]),
        ]

    pend = {0: fire_load(0)}
    ssc = {}
    for k in range(_NCH):
        p = k % 2
        for cp in pend.pop(k):
            cp.wait()
        if k - 1 >= 0:
            for cp in ssc.pop(k - 1):
                cp.wait()
        if k + 1 < _NCH:
            pend[k + 1] = fire_load(k + 1)
        ssc[k] = []
        for j in range(_KROW):
            ssc[k].append(pltpu.async_copy(vbuf.at[p, pl.ds(j * 128, 128)],
                                           shared.at[idx_a.at[p, j]], ssem,
                                           add=True))
    for cp in ssc.pop(_NCH - 1):
        cp.wait()
    plsc.subcore_barrier()
    pltpu.sync_copy(shared.at[pl.ds(s * _RPT, _RPT)], zbuf)
    pltpu.sync_copy(zbuf, out.at[c, pl.ds(s * _RPT, _RPT)])


@functools.cache
def _get_sc_scatter():
    mesh = plsc.VectorSubcoreMesh(core_axis_name="c", subcore_axis_name="s",
                                  num_cores=_NC, num_subcores=_NS)
    return pl.kernel(
        _sc_scatter_body,
        out_type=jax.ShapeDtypeStruct((_NC, _NP, _EH), jnp.float32),
        mesh=mesh,
        scratch_types=[
            pltpu.VMEM((2, _KROW, 128), jnp.int32),
            pltpu.VMEM((2, _CHUNK, _EH), jnp.float32),
            pltpu.VMEM((_RPT, _EH), jnp.float32),
            pltpu.VMEM_SHARED((_NP, _EH), jnp.float32),
            (pltpu.SemaphoreType.DMA, pltpu.SemaphoreType.DMA),
            pltpu.SemaphoreType.DMA,
        ],
        compiler_params=pltpu.CompilerParams(use_tc_tiling_on_sc=False),
    )


def _edge_body(gs, gd, ea, k3, kih, khh, ce, bre, bze, bine, bhne, out):
    a = ea[...]
    d = lambda m, w: jnp.dot(m, w[...], preferred_element_type=jnp.float32)
    eo = jnp.maximum(gs[...] + gd[...] + d(a, k3) + ce[...], 0.0)
    gi = d(eo, kih)
    gh = d(a, khh)
    r = jax.nn.sigmoid(gi[:, :128] + gh[:, :128] + bre[...])
    z = jax.nn.sigmoid(gi[:, 128:256] + gh[:, 128:256] + bze[...])
    n = jnp.tanh(gi[:, 256:] + bine[...] + r * (gh[:, 256:] + bhne[...]))
    out[...] = (1.0 - z) * n + z * a


_edge = pl.pallas_call(
    _edge_body,
    grid=(_EP // 8 // _BE,),
    in_specs=[pl.BlockSpec((_BE, 128), lambda i: (i, 0))] * 3
    + [pl.BlockSpec((128, 128), lambda i: (0, 0))]
    + [pl.BlockSpec((128, 384), lambda i: (0, 0))] * 2
    + [pl.BlockSpec((1, 128), lambda i: (0, 0))] * 5,
    out_specs=pl.BlockSpec((_BE, 128), lambda i: (i, 0)),
    out_shape=jax.ShapeDtypeStruct((_EP // 8, 128), jnp.float32),
)


def _node_body(x, aggp, cn, wn1, wn2, wihn, whhn, bihn, bhhn, xo_ref, sum_ref):
    xx = x[...]
    agg = aggp[0] + aggp[1]
    d = lambda m, w: jnp.dot(m, w[...], preferred_element_type=jnp.float32)
    xo = jnp.maximum(d(xx, wn1) + d(agg, wn2) + cn[...], 0.0)
    gi = d(xo, wihn) + bihn[...]
    gh = d(xx, whhn) + bhhn[...]
    r = jax.nn.sigmoid(gi[:, :_XH] + gh[:, :_XH])
    z = jax.nn.sigmoid(gi[:, _XH:2 * _XH] + gh[:, _XH:2 * _XH])
    n = jnp.tanh(gi[:, 2 * _XH:] + r * gh[:, 2 * _XH:])
    xn = (1.0 - z) * n + z * xx
    xo_ref[...] = xn

    @pl.when(pl.program_id(0) == 0)
    def _():
        sum_ref[...] = jnp.zeros_like(sum_ref)

    sum_ref[...] += jnp.sum(xn, axis=0, keepdims=True)


_node = pl.pallas_call(
    _node_body,
    grid=(_N // _BN,),
    in_specs=[
        pl.BlockSpec((_BN, 128), lambda i: (i, 0)),
        pl.BlockSpec((_NC, _BN, _EH), lambda i: (0, i, 0)),
        pl.BlockSpec((1, 128), lambda i: (0, 0)),
        pl.BlockSpec((128, 128), lambda i: (0, 0)),
        pl.BlockSpec((_EH, 128), lambda i: (0, 0)),
        pl.BlockSpec((128, 384), lambda i: (0, 0)),
        pl.BlockSpec((128, 384), lambda i: (0, 0)),
        pl.BlockSpec((1, 384), lambda i: (0, 0)),
        pl.BlockSpec((1, 384), lambda i: (0, 0)),
    ],
    out_specs=[
        pl.BlockSpec((_BN, 128), lambda i: (i, 0)),
        pl.BlockSpec((1, 128), lambda i: (0, 0)),
    ],
    out_shape=[
        jax.ShapeDtypeStruct((_N, 128), jnp.float32),
        jax.ShapeDtypeStruct((1, 128), jnp.float32),
    ],
)


def _prep_body(x, u, w12, w4, be, wn3, bn, ps_ref, pd_ref, ce_ref, cn_ref):
    d = lambda m, w: jnp.dot(m, w[...], preferred_element_type=jnp.float32)
    pp = d(x[...], w12)
    ps_ref[...] = pp[:, :_EH]
    pd_ref[...] = pp[:, _EH:]
    uu = u[...]
    ce_ref[...] = d(uu, w4) + be[...]
    cn_ref[...] = d(uu, wn3) + bn[...]


_prep = pl.pallas_call(
    _prep_body,
    out_shape=[
        jax.ShapeDtypeStruct((_N, _EH), jnp.float32),
        jax.ShapeDtypeStruct((_N, _EH), jnp.float32),
        jax.ShapeDtypeStruct((1, _EH), jnp.float32),
        jax.ShapeDtypeStruct((1, 128), jnp.float32),
    ],
)


def _glob_body(u, sumx, wg1, wg2, bg, wihg, whhg, bihg, bhhg,
               x, w12, w4, be, wn3, bn,
               un_ref, ps_ref, pd_ref, ce_ref, cn_ref):
    d = lambda m, w: jnp.dot(m, w[...], preferred_element_type=jnp.float32)
    uu = u[...]
    mean = sumx[...] * (1.0 / _N)
    uo = jnp.maximum(d(uu, wg1) + d(mean, wg2) + bg[...], 0.0)
    gi = d(uo, wihg) + bihg[...]
    gh = d(uu, whhg) + bhhg[...]
    r = jax.nn.sigmoid(gi[:, :_UH] + gh[:, :_UH])
    z = jax.nn.sigmoid(gi[:, _UH:2 * _UH] + gh[:, _UH:2 * _UH])
    n = jnp.tanh(gi[:, 2 * _UH:] + r * gh[:, 2 * _UH:])
    un = (1.0 - z) * n + z * uu
    un_ref[...] = un
    pp = d(x[...], w12)
    ps_ref[...] = pp[:, :_EH]
    pd_ref[...] = pp[:, _EH:]
    ce_ref[...] = d(un, w4) + be[...]
    cn_ref[...] = d(un, wn3) + bn[...]


_glob = pl.pallas_call(
    _glob_body,
    out_shape=[
        jax.ShapeDtypeStruct((1, 128), jnp.float32),
        jax.ShapeDtypeStruct((_N, _EH), jnp.float32),
        jax.ShapeDtypeStruct((_N, _EH), jnp.float32),
        jax.ShapeDtypeStruct((1, _EH), jnp.float32),
        jax.ShapeDtypeStruct((1, 128), jnp.float32),
    ],
)


def kernel(x, edge_index, edge_attr, u, batch, W_edge, b_edge, W_node, b_node,
           W_glob, b_glob, w_ih_e, w_hh_e, b_ih_e, b_hh_e, w_ih_n, w_hh_n,
           b_ih_n, b_hh_n, w_ih_g, w_hh_g, b_ih_g, b_hh_g):
    f32 = jnp.float32
    src = edge_index[0]
    dst = edge_index[1]
    pad = _EP - _E
    fill_g = jnp.arange(pad, dtype=jnp.int32) % _N
    src2 = jnp.concatenate([src, fill_g]).reshape(_RP, 128)
    dstg2 = jnp.concatenate([dst, fill_g]).reshape(_RP, 128)
    fill_s = _N + (jnp.arange(pad, dtype=jnp.int32) % (_NP - _N))
    dsts2 = jnp.concatenate([dst, fill_s]).reshape(_RP, 128)
    ea = jnp.concatenate([edge_attr.reshape(_E // 8, 128),
                          jnp.zeros((pad // 8, 128), f32)], axis=0)

    W12T = jnp.concatenate([W_edge[:, :_XH].T, W_edge[:, _XH:2 * _XH].T],
                           axis=1)
    W3T = W_edge[:, 2 * _XH:2 * _XH + _EH].T
    W4T = W_edge[:, 2 * _XH + _EH:].T
    be = b_edge[None, :]
    Wn1T = W_node[:, :_XH].T
    Wn2T = W_node[:, _XH:_XH + _EH].T
    Wn3T = W_node[:, _XH + _EH:].T
    bn = b_node[None, :]
    Wg1T = W_glob[:, :_UH].T
    Wg2T = W_glob[:, _UH:].T
    bg = b_glob[None, :]

    eye8 = jnp.eye(8, dtype=f32)
    kr = lambda w: jnp.kron(eye8, w)
    K3 = kr(W3T)
    KIH = jnp.concatenate([kr(w_ih_e[:_EH].T), kr(w_ih_e[_EH:2 * _EH].T),
                           kr(w_ih_e[2 * _EH:].T)], axis=1)
    KHH = jnp.concatenate([kr(w_hh_e[:_EH].T), kr(w_hh_e[_EH:2 * _EH].T),
                           kr(w_hh_e[2 * _EH:].T)], axis=1)
    bre = jnp.tile(b_ih_e[:_EH] + b_hh_e[:_EH], 8)[None]
    bze = jnp.tile(b_ih_e[_EH:2 * _EH] + b_hh_e[_EH:2 * _EH], 8)[None]
    bine = jnp.tile(b_ih_e[2 * _EH:], 8)[None]
    bhne = jnp.tile(b_hh_e[2 * _EH:], 8)[None]

    wihnT = w_ih_n.T
    whhnT = w_hh_n.T
    bihn = b_ih_n[None]
    bhhn = b_hh_n[None]
    wihgT = w_ih_g.T
    whhgT = w_hh_g.T
    bihg = b_ih_g[None]
    bhhg = b_hh_g[None]

    ps, pd, ce, cn = _prep(x, u, W12T, W4T, be, Wn3T, bn)
    xs = [x]
    us = [u]
    for _ in range(_STEPS):
        gs, gd = _get_sc_gather()(src2, dstg2, ps, pd)
        gs_p = gs.reshape(_EP // 8, 128)
        gd_p = gd.reshape(_EP // 8, 128)
        ce_t = jnp.tile(ce, (1, 8))
        ea = _edge(gs_p, gd_p, ea, K3, KIH, KHH, ce_t, bre, bze, bine, bhne)
        aggp = _get_sc_scatter()(dsts2, ea.reshape(_EP, _EH))
        x, sumx = _node(x, aggp, cn, Wn1T, Wn2T, wihnT, whhnT, bihn, bhhn)
        xs.append(x)
        u, ps, pd, ce, cn = _glob(u, sumx, Wg1T, Wg2T, bg, wihgT, whhgT,
                                  bihg, bhhg, x, W12T, W4T, be, Wn3T, bn)
        us.append(u)
    return jnp.concatenate(xs, axis=1), jnp.concatenate(us, axis=1)


# speedup vs baseline: 12.0240x; 1.2190x over previous
import functools

import jax
import jax.numpy as jnp
from jax import lax
from jax.experimental import pallas as pl
from jax.experimental.pallas import tpu as pltpu
from jax.experimental.pallas import tpu_sc as plsc

_N = 10000
_E = 320000
_XH = 128
_EH = 16
_UH = 128
_STEPS = 2

_NC = 2
_NS = 16
_NW = _NC * _NS

_T = 10240
_EP = _NW * _T
_RP = _EP // 128
_CHUNK = 1024
_KROW = _CHUNK // 128
_NCH = _T // _CHUNK
_TROW = _T // 128
_NP = 10240
_RPT = _NP // _NS

_BE = 512
_BN = 2000

def _sc_gather_body(srci, dsti, ps, pd, outs, outd, idx_a, idx_b, buf_s, buf_d,
                    tbuf, sh_ps, sh_pd, gsem, wsem):
    c = lax.axis_index("c")
    s = lax.axis_index("s")
    wid = s * _NC + c

    rows = _N // _NS
    pltpu.sync_copy(ps.at[pl.ds(s * rows, rows)], tbuf)
    pltpu.sync_copy(tbuf, sh_ps.at[pl.ds(s * rows, rows)])
    pltpu.sync_copy(pd.at[pl.ds(s * rows, rows)], tbuf)
    pltpu.sync_copy(tbuf, sh_pd.at[pl.ds(s * rows, rows)])
    plsc.subcore_barrier()

    def fire(k):
        p = k % 2
        row0 = wid * _TROW + k * _KROW
        pltpu.sync_copy(srci.at[pl.ds(row0, _KROW)], idx_a.at[p])
        pltpu.sync_copy(dsti.at[pl.ds(row0, _KROW)], idx_b.at[p])
        cps = []
        for j in range(_KROW):
            cps.append(pltpu.async_copy(
                sh_ps.at[idx_a.at[p, j]],
                buf_s.at[p, pl.ds(j * 128, 128)], gsem[p]))
            cps.append(pltpu.async_copy(
                sh_pd.at[idx_b.at[p, j]],
                buf_d.at[p, pl.ds(j * 128, 128)], gsem[p]))
        return cps

    pend = {0: fire(0)}
    wr = {}
    for k in range(_NCH):
        p = k % 2
        q = (k + 1) % 2
        for cp in pend.pop(k):
            cp.wait()
        row0 = wid * _TROW + k * _KROW
        wr[k] = [
            pltpu.async_copy(buf_s.at[p], outs.at[pl.ds(row0 * 128, _CHUNK)],
                             wsem[p]),
            pltpu.async_copy(buf_d.at[p], outd.at[pl.ds(row0 * 128, _CHUNK)],
                             wsem[p]),
        ]
        if k + 1 < _NCH:
            if k - 1 >= 0:
                for cp in wr.pop(k - 1):
                    cp.wait()
            pend[k + 1] = fire(k + 1)
    for k in list(wr):
        for cp in wr.pop(k):
            cp.wait()


@functools.cache
def _get_sc_gather():
    mesh = plsc.VectorSubcoreMesh(core_axis_name="c", subcore_axis_name="s",
                                  num_cores=_NC, num_subcores=_NS)
    return pl.kernel(
        _sc_gather_body,
        out_type=(jax.ShapeDtypeStruct((_EP, _EH), jnp.float32),
                  jax.ShapeDtypeStruct((_EP, _EH), jnp.float32)),
        mesh=mesh,
        scratch_types=[
            pltpu.VMEM((2, _KROW, 128), jnp.int32),
            pltpu.VMEM((2, _KROW, 128), jnp.int32),
            pltpu.VMEM((2, _CHUNK, _EH), jnp.float32),
            pltpu.VMEM((2, _CHUNK, _EH), jnp.float32),
            pltpu.VMEM((_N // _NS, _EH), jnp.float32),
            pltpu.VMEM_SHARED((_N, _EH), jnp.float32),
            pltpu.VMEM_SHARED((_N, _EH), jnp.float32),
            (pltpu.SemaphoreType.DMA, pltpu.SemaphoreType.DMA),
            (pltpu.SemaphoreType.DMA, pltpu.SemaphoreType.DMA),
        ],
        compiler_params=pltpu.CompilerParams(use_tc_tiling_on_sc=False),
    )


def _sc_scatter_body(dsti, vals, out, idx_a, vbuf, zbuf, shared, lsem, ssem):
    c = lax.axis_index("c")
    s = lax.axis_index("s")
    wid = s * _NC + c

    def zrow(i, carry):
        zbuf[i] = jnp.zeros((_EH,), jnp.float32)
        return carry

    lax.fori_loop(0, _RPT, zrow, 0)
    pltpu.sync_copy(zbuf, shared.at[pl.ds(s * _RPT, _RPT)])
    plsc.subcore_barrier()

    def fire_load(k):
        p = k % 2
        row0 = wid * _TROW + k * _KROW
        return [
            pltpu.async_copy(dsti.at[pl.ds(row0, _KROW)], idx_a.at[p],
                             lsem[p]),
            pltpu.async_copy(vals.at[pl.ds(row0 * 128, _CHUNK)], vbuf.at[p],
                             lsem[p]),
        ]

    pend = {0: fire_load(0)}
    ssc = {}
    for k in range(_NCH):
        p = k % 2
        for cp in pend.pop(k):
            cp.wait()
        if k - 1 >= 0:
            for cp in ssc.pop(k - 1):
                cp.wait()
        if k + 1 < _NCH:
            pend[k + 1] = fire_load(k + 1)
        ssc[k] = []
        for j in range(_KROW):
            ssc[k].append(pltpu.async_copy(vbuf.at[p, pl.ds(j * 128, 128)],
                                           shared.at[idx_a.at[p, j]], ssem,
                                           add=True))
    for cp in ssc.pop(_NCH - 1):
        cp.wait()
    plsc.subcore_barrier()
    pltpu.sync_copy(shared.at[pl.ds(s * _RPT, _RPT)], zbuf)
    pltpu.sync_copy(zbuf, out.at[c, pl.ds(s * _RPT, _RPT)])


@functools.cache
def _get_sc_scatter():
    mesh = plsc.VectorSubcoreMesh(core_axis_name="c", subcore_axis_name="s",
                                  num_cores=_NC, num_subcores=_NS)
    return pl.kernel(
        _sc_scatter_body,
        out_type=jax.ShapeDtypeStruct((_NC, _NP, _EH), jnp.float32),
        mesh=mesh,
        scratch_types=[
            pltpu.VMEM((2, _KROW, 128), jnp.int32),
            pltpu.VMEM((2, _CHUNK, _EH), jnp.float32),
            pltpu.VMEM((_RPT, _EH), jnp.float32),
            pltpu.VMEM_SHARED((_NP, _EH), jnp.float32),
            (pltpu.SemaphoreType.DMA, pltpu.SemaphoreType.DMA),
            pltpu.SemaphoreType.DMA,
        ],
        compiler_params=pltpu.CompilerParams(use_tc_tiling_on_sc=False),
    )


def _edge_body(gs, gd, ea, k3, kih, khh, ce, bre, bze, bine, bhne, out):
    a = ea[...]
    d = lambda m, w: jnp.dot(m, w[...], preferred_element_type=jnp.float32)
    eo = jnp.maximum(gs[...] + gd[...] + d(a, k3) + ce[...], 0.0)
    gi = d(eo, kih)
    gh = d(a, khh)
    r = jax.nn.sigmoid(gi[:, :128] + gh[:, :128] + bre[...])
    z = jax.nn.sigmoid(gi[:, 128:256] + gh[:, 128:256] + bze[...])
    n = jnp.tanh(gi[:, 256:] + bine[...] + r * (gh[:, 256:] + bhne[...]))
    out[...] = (1.0 - z) * n + z * a


def _make_edge():
    return pl.pallas_call(
        _edge_body,
        grid=(_E // 8 // 2000,),
        in_specs=[pl.BlockSpec((2000, 128), lambda i: (i, 0))] * 3
        + [pl.BlockSpec((128, 128), lambda i: (0, 0))]
        + [pl.BlockSpec((128, 384), lambda i: (0, 0))] * 2
        + [pl.BlockSpec((1, 128), lambda i: (0, 0))] * 5,
        out_specs=pl.BlockSpec((2000, 128), lambda i: (i, 0)),
        out_shape=jax.ShapeDtypeStruct((_EP // 8, 128), jnp.float32),
    )


_edge = _make_edge()


def _node_body(x, aggp, cn, wn1, wn2, wihn, whhn, bihn, bhhn, xo_ref, sum_ref):
    xx = x[...]
    agg = aggp[0] + aggp[1]
    d = lambda m, w: jnp.dot(m, w[...], preferred_element_type=jnp.float32)
    xo = jnp.maximum(d(xx, wn1) + d(agg, wn2) + cn[...], 0.0)
    gi = d(xo, wihn) + bihn[...]
    gh = d(xx, whhn) + bhhn[...]
    r = jax.nn.sigmoid(gi[:, :_XH] + gh[:, :_XH])
    z = jax.nn.sigmoid(gi[:, _XH:2 * _XH] + gh[:, _XH:2 * _XH])
    n = jnp.tanh(gi[:, 2 * _XH:] + r * gh[:, 2 * _XH:])
    xn = (1.0 - z) * n + z * xx
    xo_ref[...] = xn

    @pl.when(pl.program_id(0) == 0)
    def _():
        sum_ref[...] = jnp.zeros_like(sum_ref)

    sum_ref[...] += jnp.sum(xn, axis=0, keepdims=True)


_node = pl.pallas_call(
    _node_body,
    grid=(_N // _BN,),
    in_specs=[
        pl.BlockSpec((_BN, 128), lambda i: (i, 0)),
        pl.BlockSpec((_NC, _BN, _EH), lambda i: (0, i, 0)),
        pl.BlockSpec((1, 128), lambda i: (0, 0)),
        pl.BlockSpec((128, 128), lambda i: (0, 0)),
        pl.BlockSpec((_EH, 128), lambda i: (0, 0)),
        pl.BlockSpec((128, 384), lambda i: (0, 0)),
        pl.BlockSpec((128, 384), lambda i: (0, 0)),
        pl.BlockSpec((1, 384), lambda i: (0, 0)),
        pl.BlockSpec((1, 384), lambda i: (0, 0)),
    ],
    out_specs=[
        pl.BlockSpec((_BN, 128), lambda i: (i, 0)),
        pl.BlockSpec((1, 128), lambda i: (0, 0)),
    ],
    out_shape=[
        jax.ShapeDtypeStruct((_N, 128), jnp.float32),
        jax.ShapeDtypeStruct((1, 128), jnp.float32),
    ],
)


def _prep_body(x, u, w12, w4, be, wn3, bn, ps_ref, pd_ref, ce_ref, cn_ref):
    d = lambda m, w: jnp.dot(m, w[...], preferred_element_type=jnp.float32)
    pp = d(x[...], w12)
    ps_ref[...] = pp[:, :_EH]
    pd_ref[...] = pp[:, _EH:]
    uu = u[...]
    ce_ref[...] = d(uu, w4) + be[...]
    cn_ref[...] = d(uu, wn3) + bn[...]


_prep = pl.pallas_call(
    _prep_body,
    out_shape=[
        jax.ShapeDtypeStruct((_N, _EH), jnp.float32),
        jax.ShapeDtypeStruct((_N, _EH), jnp.float32),
        jax.ShapeDtypeStruct((1, _EH), jnp.float32),
        jax.ShapeDtypeStruct((1, 128), jnp.float32),
    ],
)


def _glob_body(u, sumx, wg1, wg2, bg, wihg, whhg, bihg, bhhg,
               x, w12, w4, be, wn3, bn,
               un_ref, ps_ref, pd_ref, ce_ref, cn_ref):
    d = lambda m, w: jnp.dot(m, w[...], preferred_element_type=jnp.float32)
    uu = u[...]
    mean = sumx[...] * (1.0 / _N)
    uo = jnp.maximum(d(uu, wg1) + d(mean, wg2) + bg[...], 0.0)
    gi = d(uo, wihg) + bihg[...]
    gh = d(uu, whhg) + bhhg[...]
    r = jax.nn.sigmoid(gi[:, :_UH] + gh[:, :_UH])
    z = jax.nn.sigmoid(gi[:, _UH:2 * _UH] + gh[:, _UH:2 * _UH])
    n = jnp.tanh(gi[:, 2 * _UH:] + r * gh[:, 2 * _UH:])
    un = (1.0 - z) * n + z * uu
    un_ref[...] = un
    pp = d(x[...], w12)
    ps_ref[...] = pp[:, :_EH]
    pd_ref[...] = pp[:, _EH:]
    ce_ref[...] = d(un, w4) + be[...]
    cn_ref[...] = d(un, wn3) + bn[...]


_glob = pl.pallas_call(
    _glob_body,
    out_shape=[
        jax.ShapeDtypeStruct((1, 128), jnp.float32),
        jax.ShapeDtypeStruct((_N, _EH), jnp.float32),
        jax.ShapeDtypeStruct((_N, _EH), jnp.float32),
        jax.ShapeDtypeStruct((1, _EH), jnp.float32),
        jax.ShapeDtypeStruct((1, 128), jnp.float32),
    ],
)


def kernel(x, edge_index, edge_attr, u, batch, W_edge, b_edge, W_node, b_node,
           W_glob, b_glob, w_ih_e, w_hh_e, b_ih_e, b_hh_e, w_ih_n, w_hh_n,
           b_ih_n, b_hh_n, w_ih_g, w_hh_g, b_ih_g, b_hh_g):
    f32 = jnp.float32
    src = edge_index[0]
    dst = edge_index[1]
    pad = _EP - _E
    fill_g = jnp.arange(pad, dtype=jnp.int32) % _N
    src2 = jnp.concatenate([src, fill_g]).reshape(_RP, 128)
    dstg2 = jnp.concatenate([dst, fill_g]).reshape(_RP, 128)
    fill_s = _N + (jnp.arange(pad, dtype=jnp.int32) % (_NP - _N))
    dsts2 = jnp.concatenate([dst, fill_s]).reshape(_RP, 128)
    ea = edge_attr.reshape(_E // 8, 128)

    W12T = jnp.concatenate([W_edge[:, :_XH].T, W_edge[:, _XH:2 * _XH].T],
                           axis=1)
    W3T = W_edge[:, 2 * _XH:2 * _XH + _EH].T
    W4T = W_edge[:, 2 * _XH + _EH:].T
    be = b_edge[None, :]
    Wn1T = W_node[:, :_XH].T
    Wn2T = W_node[:, _XH:_XH + _EH].T
    Wn3T = W_node[:, _XH + _EH:].T
    bn = b_node[None, :]
    Wg1T = W_glob[:, :_UH].T
    Wg2T = W_glob[:, _UH:].T
    bg = b_glob[None, :]

    eye8 = jnp.eye(8, dtype=f32)
    kr = lambda w: jnp.kron(eye8, w)
    K3 = kr(W3T)
    KIH = jnp.concatenate([kr(w_ih_e[:_EH].T), kr(w_ih_e[_EH:2 * _EH].T),
                           kr(w_ih_e[2 * _EH:].T)], axis=1)
    KHH = jnp.concatenate([kr(w_hh_e[:_EH].T), kr(w_hh_e[_EH:2 * _EH].T),
                           kr(w_hh_e[2 * _EH:].T)], axis=1)
    bre = jnp.tile(b_ih_e[:_EH] + b_hh_e[:_EH], 8)[None]
    bze = jnp.tile(b_ih_e[_EH:2 * _EH] + b_hh_e[_EH:2 * _EH], 8)[None]
    bine = jnp.tile(b_ih_e[2 * _EH:], 8)[None]
    bhne = jnp.tile(b_hh_e[2 * _EH:], 8)[None]

    wihnT = w_ih_n.T
    whhnT = w_hh_n.T
    bihn = b_ih_n[None]
    bhhn = b_hh_n[None]
    wihgT = w_ih_g.T
    whhgT = w_hh_g.T
    bihg = b_ih_g[None]
    bhhg = b_hh_g[None]

    ps, pd, ce, cn = _prep(x, u, W12T, W4T, be, Wn3T, bn)
    xs = [x]
    us = [u]
    for _ in range(_STEPS):
        gs, gd = _get_sc_gather()(src2, dstg2, ps, pd)
        gs_p = gs.reshape(_EP // 8, 128)
        gd_p = gd.reshape(_EP // 8, 128)
        ce_t = jnp.tile(ce, (1, 8))
        ea = _edge(gs_p, gd_p, ea, K3, KIH, KHH, ce_t, bre, bze, bine, bhne)
        aggp = _get_sc_scatter()(dsts2, ea.reshape(_EP, _EH))
        x, sumx = _node(x, aggp, cn, Wn1T, Wn2T, wihnT, whhnT, bihn, bhhn)
        xs.append(x)
        u, ps, pd, ce, cn = _glob(u, sumx, Wg1T, Wg2T, bg, wihgT, whhgT,
                                  bihg, bhhg, x, W12T, W4T, be, Wn3T, bn)
        us.append(u)
    return jnp.concatenate(xs, axis=1), jnp.concatenate(us, axis=1)
